# baseline (speedup 1.0000x reference)
import jax
import jax.numpy as jnp
from jax.experimental import pallas as pl
from jax.experimental.pallas import tpu as pltpu

_N = 5000
_C = 81
_M = 28


def _body(lab_ref, x_ref, o_ref):
    o_ref[...] = jax.nn.sigmoid(x_ref[...])


def kernel(x, labels):
    labels32 = labels.astype(jnp.int32)
    out = pl.pallas_call(
        _body,
        grid_spec=pltpu.PrefetchScalarGridSpec(
            num_scalar_prefetch=1,
            grid=(_N,),
            in_specs=[
                pl.BlockSpec((1, 1, _M, _M), lambda i, lab: (i, lab[i], 0, 0)),
            ],
            out_specs=pl.BlockSpec((1, 1, _M, _M), lambda i, lab: (i, 0, 0, 0)),
        ),
        out_shape=jax.ShapeDtypeStruct((_N, 1, _M, _M), jnp.float32),
    )(labels32, x)
    return out


# speedup vs baseline: 1.4202x; 1.4202x over previous
import jax
import jax.numpy as jnp
from jax import lax
from jax.experimental import pallas as pl
from jax.experimental.pallas import tpu as pltpu
from jax.experimental.pallas import tpu_sc as plsc

_N = 5000
_C = 81
_M = 28
_NC = 2
_NS = 16
_NW = _NC * _NS
_CH = 8
_NCHUNKS = (_N + _CH - 1) // _CH
_CPW = (_NCHUNKS + _NW - 1) // _NW
_LABPAD = 5136
_NEG = -2147483648


def _sigmoid(v):
    return 1.0 / (1.0 + jnp.exp(-v))


def _sc_body(x_hbm, lab_hbm, out_hbm, lab_v, *rest):
    bufs = rest[:2 * _CH]
    gsem, osem = rest[2 * _CH], rest[2 * _CH + 1]
    wid = lax.axis_index("s") * _NC + lax.axis_index("c")
    pltpu.sync_copy(lab_hbm.at[pl.ds(wid * (_CPW * _CH), _CPW * _CH + 16)], lab_v)
    iot = lax.iota(jnp.int32, 16)

    def issue_gathers(j, h):
        base = (wid * _CPW + j) * _CH
        lab16 = lab_v[pl.ds(j * _CH, 16)]
        for k in range(_CH):
            labk = jnp.max(jnp.where(iot == k, lab16, _NEG))
            row = jnp.minimum(base + k, _N - 1)
            pltpu.async_copy(x_hbm.at[row, labk], bufs[h * _CH + k], gsem)

    def wait_gathers(h):
        for k in range(_CH):
            pltpu.make_async_copy(x_hbm.at[0, 0], bufs[h * _CH + k], gsem).wait()

    def issue_outs(j, h):
        base = (wid * _CPW + j) * _CH
        for k in range(_CH):
            @pl.when(base + k < _N)
            def _():
                pltpu.async_copy(bufs[h * _CH + k], out_hbm.at[base + k], osem)

    def wait_outs(j, h):
        base = (wid * _CPW + j) * _CH
        for k in range(_CH):
            @pl.when(base + k < _N)
            def _():
                pltpu.make_async_copy(bufs[h * _CH + k],
                                      out_hbm.at[base + k], osem).wait()

    def sigmoid_half(h):
        def body(r, c):
            for k in range(_CH):
                b = bufs[h * _CH + k]
                v0 = b[r, pl.ds(0, 16)]
                v1 = b[r, pl.ds(12, 16)]
                b[r, pl.ds(0, 16)] = _sigmoid(v0)
                b[r, pl.ds(12, 16)] = _sigmoid(v1)
            return c
        lax.fori_loop(0, _M, body, 0)

    @pl.when(wid * _CPW < _NCHUNKS)
    def _():
        issue_gathers(0, 0)

    def iter_body(j, carry):
        chunk = wid * _CPW + j
        valid = chunk < _NCHUNKS

        def do(h):
            @pl.when(valid)
            def _():
                wait_gathers(h)

            @pl.when((j > 0) & (chunk <= _NCHUNKS))
            def _():
                wait_outs(j - 1, 1 - h)

            @pl.when((j < _CPW - 1) & (chunk + 1 < _NCHUNKS))
            def _():
                issue_gathers(j + 1, 1 - h)

            @pl.when(valid)
            def _():
                sigmoid_half(h)
                issue_outs(j, h)

        @pl.when(j % 2 == 0)
        def _():
            do(0)

        @pl.when(j % 2 == 1)
        def _():
            do(1)

        return carry

    lax.fori_loop(0, _CPW, iter_body, 0)

    @pl.when(wid * _CPW + _CPW - 1 < _NCHUNKS)
    def _():
        wait_outs(_CPW - 1, (_CPW - 1) % 2)


def kernel(x, labels):
    lab = jnp.zeros((_LABPAD,), jnp.int32).at[:_N].set(labels.astype(jnp.int32))
    mesh = plsc.VectorSubcoreMesh(core_axis_name="c", subcore_axis_name="s")
    k = pl.kernel(
        _sc_body,
        out_type=jax.ShapeDtypeStruct((_N, _M, _M), jnp.float32),
        mesh=mesh,
        compiler_params=pltpu.CompilerParams(needs_layout_passes=False),
        scratch_types=[pltpu.VMEM((_CPW * _CH + 16,), jnp.int32)]
        + [pltpu.VMEM((_M, _M), jnp.float32) for _ in range(2 * _CH)]
        + [pltpu.SemaphoreType.DMA, pltpu.SemaphoreType.DMA],
    )
    out = k(x, lab)
    return out.reshape(_N, 1, _M, _M)


# speedup vs baseline: 6.5390x; 4.6042x over previous
import jax
import jax.numpy as jnp
from jax import lax
from jax.experimental import pallas as pl
from jax.experimental.pallas import tpu as pltpu
from jax.experimental.pallas import tpu_sc as plsc

_N = 5000
_C = 81
_M = 28
_P = _M * _M
_NC = 2
_NS = 16
_NW = _NC * _NS
_CW = 128
_NCHUNK = (_N // _CW)
_NCOLS = _NCHUNK * _CW
_PG = 7
_PPG = _P // _PG
_NU = _NCHUNK * _PG
_UPW = (_NU + _NW - 1) // _NW


def _sigmoid(v):
    return 1.0 / (1.0 + jnp.exp(-v))


def _sc_body(x_hbm, lab_hbm, out_hbm, lab_v, a0, a1, obuf, gsem, osem):
    wid = lax.axis_index("s") * _NC + lax.axis_index("c")
    iot = lax.iota(jnp.int32, 16)
    abufs = (a0, a1)

    def unit_body(u, carry):
        unit = wid + u * _NW

        @pl.when(unit < _NU)
        def _():
            cc = unit % _NCHUNK
            pg = unit // _NCHUNK
            col0 = pl.multiple_of(cc * _CW, _CW)
            p0 = pl.multiple_of(pg * _PPG, _PPG)
            pltpu.sync_copy(lab_hbm.at[pl.ds(col0, _CW)], lab_v)

            def fetch(pp, buf):
                pltpu.async_copy(x_hbm.at[p0 + pp, :, pl.ds(col0, _CW)],
                                 buf, gsem)

            def wait_fetch(buf):
                pltpu.make_async_copy(x_hbm.at[0, :, pl.ds(0, _CW)],
                                      buf, gsem).wait()

            fetch(0, abufs[0])

            def plane_body(pp, c2):
                def run(h):
                    buf = abufs[h]
                    wait_fetch(buf)

                    @pl.when(pp + 1 < _PPG)
                    def _():
                        fetch(pp + 1, abufs[1 - h])

                    for g in range(_CW // 16):
                        rows = lab_v[pl.ds(g * 16, 16)]
                        cols = g * 16 + iot
                        v = plsc.load_gather(buf, [rows, cols])
                        obuf[pp, pl.ds(g * 16, 16)] = _sigmoid(v)

                @pl.when(pp % 2 == 0)
                def _():
                    run(0)

                @pl.when(pp % 2 == 1)
                def _():
                    run(1)

                return c2

            lax.fori_loop(0, _PPG, plane_body, 0)
            pltpu.async_copy(obuf,
                             out_hbm.at[pl.ds(p0, _PPG), pl.ds(col0, _CW)],
                             osem)
            pltpu.make_async_copy(obuf,
                                  out_hbm.at[pl.ds(0, _PPG), pl.ds(0, _CW)],
                                  osem).wait()

        return carry

    lax.fori_loop(0, _UPW, unit_body, 0)


def kernel(x, labels):
    lab32 = labels.astype(jnp.int32)
    xt = jnp.transpose(x, (2, 3, 1, 0)).reshape(_P, _C, _N)
    mesh = plsc.VectorSubcoreMesh(core_axis_name="c", subcore_axis_name="s")
    k = pl.kernel(
        _sc_body,
        out_type=jax.ShapeDtypeStruct((_P, _NCOLS), jnp.float32),
        mesh=mesh,
        compiler_params=pltpu.CompilerParams(needs_layout_passes=False),
        scratch_types=[
            pltpu.VMEM((_CW,), jnp.int32),
            pltpu.VMEM((_C, _CW), jnp.float32),
            pltpu.VMEM((_C, _CW), jnp.float32),
            pltpu.VMEM((_PPG, _CW), jnp.float32),
            pltpu.SemaphoreType.DMA,
            pltpu.SemaphoreType.DMA,
        ],
    )
    out_t = k(xt, lab32)
    main = jnp.transpose(out_t.reshape(_M, _M, _NCOLS), (2, 0, 1))
    xtail = x[_NCOLS:]
    sel = jnp.take_along_axis(
        xtail, labels[_NCOLS:].astype(jnp.int32)[:, None, None, None],
        axis=1)[:, 0]
    tail = jax.nn.sigmoid(sel)
    out = jnp.concatenate([main, tail], axis=0)
    return out[:, None]


# speedup vs baseline: 13.5295x; 2.0691x over previous
import jax
import jax.numpy as jnp
from jax import lax
from jax.experimental import pallas as pl
from jax.experimental.pallas import tpu as pltpu
from jax.experimental.pallas import tpu_sc as plsc

_N = 5000
_C = 81
_M = 28
_P = _M * _M
_NC = 2
_NS = 16
_NW = _NC * _NS
_CW = 128
_NCHUNK = (_N // _CW)
_NCOLS = _NCHUNK * _CW
_PG = 7
_PPG = _P // _PG
_NU = _NCHUNK * _PG
_UPW = (_NU + _NW - 1) // _NW


def _sigmoid(v):
    return 1.0 / (1.0 + jnp.exp(-v))


def _sc_body(x_hbm, lab_hbm, out_hbm, lab_v, a0, a1, a2, a3, obuf, gsem, osem):
    wid = lax.axis_index("s") * _NC + lax.axis_index("c")
    iot = lax.iota(jnp.int32, 16)
    abufs = (a0, a1, a2, a3)

    def unit_body(u, carry):
        unit = wid + u * _NW

        @pl.when(unit < _NU)
        def _():
            cc = unit % _NCHUNK
            pg = unit // _NCHUNK
            col0 = pl.multiple_of(cc * _CW, _CW)
            p0 = pl.multiple_of(pg * _PPG, _PPG)
            pltpu.sync_copy(lab_hbm.at[pl.ds(col0, _CW)], lab_v)

            def fetch(pp, buf):
                pltpu.async_copy(x_hbm.at[p0 + pp, :, pl.ds(col0, _CW)],
                                 buf, gsem)

            def wait_fetch(buf):
                pltpu.make_async_copy(x_hbm.at[0, :, pl.ds(0, _CW)],
                                      buf, gsem).wait()

            fetch(0, abufs[0])
            fetch(1, abufs[1])
            fetch(2, abufs[2])

            def plane_body(pp, c2):
                def run(h):
                    buf = abufs[h]
                    wait_fetch(buf)

                    @pl.when(pp + 3 < _PPG)
                    def _():
                        fetch(pp + 3, abufs[(h + 3) % 4])

                    for g in range(_CW // 16):
                        rows = lab_v[pl.ds(g * 16, 16)]
                        cols = g * 16 + iot
                        v = plsc.load_gather(buf, [rows, cols])
                        obuf[pp, pl.ds(g * 16, 16)] = _sigmoid(v)

                for h in range(4):
                    @pl.when(pp % 4 == h)
                    def _(h=h):
                        run(h)

                return c2

            lax.fori_loop(0, _PPG, plane_body, 0)
            pltpu.async_copy(obuf,
                             out_hbm.at[pl.ds(p0, _PPG), pl.ds(col0, _CW)],
                             osem)
            pltpu.make_async_copy(obuf,
                                  out_hbm.at[pl.ds(0, _PPG), pl.ds(0, _CW)],
                                  osem).wait()

        return carry

    lax.fori_loop(0, _UPW, unit_body, 0)


def kernel(x, labels):
    lab32 = labels.astype(jnp.int32)
    xt = jnp.transpose(x, (2, 3, 1, 0)).reshape(_P, _C, _N)
    mesh = plsc.VectorSubcoreMesh(core_axis_name="c", subcore_axis_name="s")
    k = pl.kernel(
        _sc_body,
        out_type=jax.ShapeDtypeStruct((_P, _NCOLS), jnp.float32),
        mesh=mesh,
        compiler_params=pltpu.CompilerParams(needs_layout_passes=False),
        scratch_types=[
            pltpu.VMEM((_CW,), jnp.int32),
            pltpu.VMEM((_C, _CW), jnp.float32),
            pltpu.VMEM((_C, _CW), jnp.float32),
            pltpu.VMEM((_C, _CW), jnp.float32),
            pltpu.VMEM((_C, _CW), jnp.float32),
            pltpu.VMEM((_PPG, _CW), jnp.float32),
            pltpu.SemaphoreType.DMA,
            pltpu.SemaphoreType.DMA,
        ],
    )
    out_t = k(xt, lab32)
    main = jnp.transpose(out_t.reshape(_M, _M, _NCOLS), (2, 0, 1))
    xtail = x[_NCOLS:]
    sel = jnp.take_along_axis(
        xtail, labels[_NCOLS:].astype(jnp.int32)[:, None, None, None],
        axis=1)[:, 0]
    tail = jax.nn.sigmoid(sel)
    out = jnp.concatenate([main, tail], axis=0)
    return out[:, None]


# speedup vs baseline: 14.5612x; 1.0763x over previous
import jax
import jax.numpy as jnp
from jax import lax
from jax.experimental import pallas as pl
from jax.experimental.pallas import tpu as pltpu
from jax.experimental.pallas import tpu_sc as plsc

_N = 5000
_C = 81
_M = 28
_P = _M * _M
_NC = 2
_NS = 16
_NW = _NC * _NS
_CW = 128
_NCHUNK = (_N // _CW)
_NCOLS = _NCHUNK * _CW
_PG = 7
_PPG = _P // _PG
_NU = _NCHUNK * _PG
_UPW = (_NU + _NW - 1) // _NW


def _sigmoid(v):
    return 1.0 / (1.0 + jnp.exp(-v))


def _sc_body(x_hbm, lab_hbm, out_hbm, lab_v, a0, a1, a2, a3, a4, a5, obuf, gsem, osem):
    wid = lax.axis_index("s") * _NC + lax.axis_index("c")
    iot = lax.iota(jnp.int32, 16)
    abufs = (a0, a1, a2, a3, a4, a5)

    def unit_body(u, carry):
        unit = wid + u * _NW

        @pl.when(unit < _NU)
        def _():
            cc = unit % _NCHUNK
            pg = unit // _NCHUNK
            col0 = pl.multiple_of(cc * _CW, _CW)
            p0 = pl.multiple_of(pg * _PPG, _PPG)
            pltpu.sync_copy(lab_hbm.at[pl.ds(col0, _CW)], lab_v)

            def fetch(pp, buf):
                pltpu.async_copy(x_hbm.at[p0 + pp, :, pl.ds(col0, _CW)],
                                 buf, gsem)

            def wait_fetch(buf):
                pltpu.make_async_copy(x_hbm.at[0, :, pl.ds(0, _CW)],
                                      buf, gsem).wait()

            for pr in range(5):
                fetch(pr, abufs[pr])

            def plane_body(pp, c2):
                def run(h):
                    buf = abufs[h]
                    wait_fetch(buf)

                    @pl.when(pp + 5 < _PPG)
                    def _():
                        fetch(pp + 5, abufs[(h + 5) % 6])

                    for g in range(_CW // 16):
                        rows = lab_v[pl.ds(g * 16, 16)]
                        cols = g * 16 + iot
                        v = plsc.load_gather(buf, [rows, cols])
                        obuf[pp, pl.ds(g * 16, 16)] = _sigmoid(v)

                for h in range(6):
                    @pl.when(pp % 6 == h)
                    def _(h=h):
                        run(h)

                return c2

            lax.fori_loop(0, _PPG, plane_body, 0)
            pltpu.async_copy(obuf,
                             out_hbm.at[pl.ds(p0, _PPG), pl.ds(col0, _CW)],
                             osem)
            pltpu.make_async_copy(obuf,
                                  out_hbm.at[pl.ds(0, _PPG), pl.ds(0, _CW)],
                                  osem).wait()

        return carry

    lax.fori_loop(0, _UPW, unit_body, 0)


def kernel(x, labels):
    lab32 = labels.astype(jnp.int32)
    xt = jnp.transpose(x, (2, 3, 1, 0)).reshape(_P, _C, _N)
    mesh = plsc.VectorSubcoreMesh(core_axis_name="c", subcore_axis_name="s")
    k = pl.kernel(
        _sc_body,
        out_type=jax.ShapeDtypeStruct((_P, _NCOLS), jnp.float32),
        mesh=mesh,
        compiler_params=pltpu.CompilerParams(needs_layout_passes=False),
        scratch_types=[
            pltpu.VMEM((_CW,), jnp.int32),
            pltpu.VMEM((_C, _CW), jnp.float32),
            pltpu.VMEM((_C, _CW), jnp.float32),
            pltpu.VMEM((_C, _CW), jnp.float32),
            pltpu.VMEM((_C, _CW), jnp.float32),
            pltpu.VMEM((_C, _CW), jnp.float32),
            pltpu.VMEM((_C, _CW), jnp.float32),
            pltpu.VMEM((_PPG, _CW), jnp.float32),
            pltpu.SemaphoreType.DMA,
            pltpu.SemaphoreType.DMA,
        ],
    )
    out_t = k(xt, lab32)
    main = jnp.transpose(out_t.reshape(_M, _M, _NCOLS), (2, 0, 1))
    xtail = x[_NCOLS:]
    sel = jnp.take_along_axis(
        xtail, labels[_NCOLS:].astype(jnp.int32)[:, None, None, None],
        axis=1)[:, 0]
    tail = jax.nn.sigmoid(sel)
    out = jnp.concatenate([main, tail], axis=0)
    return out[:, None]
